# baseline (speedup 1.0000x reference)
import functools

import jax
import jax.numpy as jnp
from jax import lax
from jax.experimental import pallas as pl
from jax.experimental.pallas import tpu as pltpu
from jax.experimental.pallas import tpu_sc as plsc

_NUM_NODES = 1000000
_EMBED_DIM = 64
_NUM_CORES = 2
_NUM_SUBCORES = 16
_NUM_WORKERS = _NUM_CORES * _NUM_SUBCORES

_BC = 8192
_SC_COLS = 417792
_NQ = 4
_COLS_PER_W = _SC_COLS // _NQ
_CHUNK = 3072
_NCHUNKS = _COLS_PER_W // _CHUNK
_NBUF = 4
_NBANDS = 8

_TC_GRID = (_NUM_NODES - _SC_COLS + _BC - 1) // _BC

_MESH = plsc.VectorSubcoreMesh(core_axis_name="c", subcore_axis_name="s")


@functools.partial(
    pl.kernel,
    out_type=jax.ShapeDtypeStruct((_EMBED_DIM, _SC_COLS), jnp.float32),
    mesh=_MESH,
    scratch_types=[
        [pltpu.VMEM((8, _CHUNK), jnp.float32) for _ in range(_NBUF)],
        [pltpu.SemaphoreType.DMA for _ in range(_NBUF)],
        [pltpu.SemaphoreType.DMA for _ in range(_NBUF)],
    ],
)
def _sc_copy(w_hbm, o_hbm, bufs, in_sems, out_sems):
    wid = lax.axis_index("s") * _NUM_CORES + lax.axis_index("c")
    band = wid // _NQ
    row = pl.multiple_of(band * 8, 8)
    cbase = pl.multiple_of((wid % _NQ) * _COLS_PER_W, 128)

    def _in_copy(k, b):
        off = pl.multiple_of(cbase + k * _CHUNK, 128)
        return pltpu.make_async_copy(
            w_hbm.at[pl.ds(row, 8), pl.ds(off, _CHUNK)], bufs[b], in_sems[b])

    def _out_copy(k, b):
        off = pl.multiple_of(cbase + k * _CHUNK, 128)
        return pltpu.make_async_copy(
            bufs[b], o_hbm.at[pl.ds(row, 8), pl.ds(off, _CHUNK)], out_sems[b])

    for j in range(_NBUF - 1):
        _in_copy(j, j).start()
    for k in range(_NCHUNKS):
        b = k % _NBUF
        _in_copy(k, b).wait()
        _out_copy(k, b).start()
        if k + _NBUF - 1 < _NCHUNKS:
            if k >= 1:
                _out_copy(k - 1, (k + _NBUF - 1) % _NBUF).wait()
            _in_copy(k + _NBUF - 1, (k + _NBUF - 1) % _NBUF).start()
    for k in range(_NCHUNKS - _NBUF, _NCHUNKS):
        _out_copy(k, k % _NBUF).wait()


def _tc_body(w_ref, o_ref):
    o_ref[...] = w_ref[...]


def kernel(weight):
    wt = weight.T
    out_sc = _sc_copy(wt)
    out_tc = pl.pallas_call(
        _tc_body,
        out_shape=jax.ShapeDtypeStruct((_EMBED_DIM, _NUM_NODES), jnp.float32),
        grid=(_TC_GRID,),
        in_specs=[pl.BlockSpec((_EMBED_DIM, _BC),
                               lambda i: (0, i + _SC_COLS // _BC))],
        out_specs=pl.BlockSpec((_EMBED_DIM, _BC),
                               lambda i: (0, i + _SC_COLS // _BC)),
    )(wt)
    out_t = lax.dynamic_update_slice(out_tc, out_sc, (0, 0))
    return out_t.T


# speedup vs baseline: 1.2790x; 1.2790x over previous
import functools

import jax
import jax.numpy as jnp
from jax import lax
from jax.experimental import pallas as pl
from jax.experimental.pallas import tpu as pltpu
from jax.experimental.pallas import tpu_sc as plsc

_NUM_NODES = 1000000
_EMBED_DIM = 64
_NUM_CORES = 2
_NUM_SUBCORES = 16
_NUM_WORKERS = _NUM_CORES * _NUM_SUBCORES

_BC = 8192
_SC_COLS = 417792
_NQ = 4
_COLS_PER_W = _SC_COLS // _NQ
_CHUNK = 3072
_NCHUNKS = _COLS_PER_W // _CHUNK
_NBUF = 4
_NBANDS = 8

_TC_GRID = (_NUM_NODES - _SC_COLS + _BC - 1) // _BC

_MESH = plsc.VectorSubcoreMesh(core_axis_name="c", subcore_axis_name="s")


@functools.partial(
    pl.kernel,
    out_type=jax.ShapeDtypeStruct((_EMBED_DIM, _NUM_NODES), jnp.float32),
    mesh=_MESH,
    scratch_types=[
        [pltpu.VMEM((8, _CHUNK), jnp.float32) for _ in range(_NBUF)],
        [pltpu.SemaphoreType.DMA for _ in range(_NBUF)],
        [pltpu.SemaphoreType.DMA for _ in range(_NBUF)],
    ],
)
def _sc_copy(w_hbm, o_hbm, bufs, in_sems, out_sems):
    wid = lax.axis_index("s") * _NUM_CORES + lax.axis_index("c")
    band = wid // _NQ
    row = pl.multiple_of(band * 8, 8)
    cbase = pl.multiple_of((wid % _NQ) * _COLS_PER_W, 128)

    def _in_copy(k, b):
        off = pl.multiple_of(cbase + k * _CHUNK, 128)
        return pltpu.make_async_copy(
            w_hbm.at[pl.ds(row, 8), pl.ds(off, _CHUNK)], bufs[b], in_sems[b])

    def _out_copy(k, b):
        off = pl.multiple_of(cbase + k * _CHUNK, 128)
        return pltpu.make_async_copy(
            bufs[b], o_hbm.at[pl.ds(row, 8), pl.ds(off, _CHUNK)], out_sems[b])

    for j in range(_NBUF - 1):
        _in_copy(j, j).start()
    for k in range(_NCHUNKS):
        b = k % _NBUF
        _in_copy(k, b).wait()
        _out_copy(k, b).start()
        if k + _NBUF - 1 < _NCHUNKS:
            if k >= 1:
                _out_copy(k - 1, (k + _NBUF - 1) % _NBUF).wait()
            _in_copy(k + _NBUF - 1, (k + _NBUF - 1) % _NBUF).start()
    for k in range(_NCHUNKS - _NBUF, _NCHUNKS):
        _out_copy(k, k % _NBUF).wait()


def _tc_body(w_ref, acc_ref, o_ref):
    o_ref[...] = w_ref[...]


def kernel(weight):
    wt = weight.T
    out_sc = _sc_copy(wt)
    out_t = pl.pallas_call(
        _tc_body,
        out_shape=jax.ShapeDtypeStruct((_EMBED_DIM, _NUM_NODES), jnp.float32),
        grid=(_TC_GRID,),
        in_specs=[
            pl.BlockSpec((_EMBED_DIM, _BC),
                         lambda i: (0, i + _SC_COLS // _BC)),
            pl.BlockSpec(memory_space=pl.ANY),
        ],
        out_specs=pl.BlockSpec((_EMBED_DIM, _BC),
                               lambda i: (0, i + _SC_COLS // _BC)),
        input_output_aliases={1: 0},
    )(wt, out_sc)
    return out_t.T


# speedup vs baseline: 1.3406x; 1.0482x over previous
import functools

import jax
import jax.numpy as jnp
from jax import lax
from jax.experimental import pallas as pl
from jax.experimental.pallas import tpu as pltpu
from jax.experimental.pallas import tpu_sc as plsc

_NUM_NODES = 1000000
_EMBED_DIM = 64
_NUM_CORES = 2
_NUM_SUBCORES = 16
_NUM_WORKERS = _NUM_CORES * _NUM_SUBCORES

_BC = 16384
_SC_COLS = 409600
_NQ = 4
_COLS_PER_W = _SC_COLS // _NQ
_CHUNK = 3200
_NCHUNKS = _COLS_PER_W // _CHUNK
_NBUF = 4
_NBANDS = 8

_TC_GRID = (_NUM_NODES - _SC_COLS + _BC - 1) // _BC

_MESH = plsc.VectorSubcoreMesh(core_axis_name="c", subcore_axis_name="s")


@functools.partial(
    pl.kernel,
    out_type=jax.ShapeDtypeStruct((_EMBED_DIM, _NUM_NODES), jnp.float32),
    mesh=_MESH,
    scratch_types=[
        [pltpu.VMEM((8, _CHUNK), jnp.float32) for _ in range(_NBUF)],
        [pltpu.SemaphoreType.DMA for _ in range(_NBUF)],
        [pltpu.SemaphoreType.DMA for _ in range(_NBUF)],
    ],
)
def _sc_copy(w_hbm, o_hbm, bufs, in_sems, out_sems):
    wid = lax.axis_index("s") * _NUM_CORES + lax.axis_index("c")
    band = wid // _NQ
    row = pl.multiple_of(band * 8, 8)
    cbase = pl.multiple_of((wid % _NQ) * _COLS_PER_W, 128)

    def _in_copy(k, b):
        off = pl.multiple_of(cbase + k * _CHUNK, 128)
        return pltpu.make_async_copy(
            w_hbm.at[pl.ds(row, 8), pl.ds(off, _CHUNK)], bufs[b], in_sems[b])

    def _out_copy(k, b):
        off = pl.multiple_of(cbase + k * _CHUNK, 128)
        return pltpu.make_async_copy(
            bufs[b], o_hbm.at[pl.ds(row, 8), pl.ds(off, _CHUNK)], out_sems[b])

    for j in range(_NBUF - 1):
        _in_copy(j, j).start()
    for k in range(_NCHUNKS):
        b = k % _NBUF
        _in_copy(k, b).wait()
        _out_copy(k, b).start()
        if k + _NBUF - 1 < _NCHUNKS:
            if k >= 1:
                _out_copy(k - 1, (k + _NBUF - 1) % _NBUF).wait()
            _in_copy(k + _NBUF - 1, (k + _NBUF - 1) % _NBUF).start()
    for k in range(_NCHUNKS - _NBUF, _NCHUNKS):
        _out_copy(k, k % _NBUF).wait()


def _tc_body(w_ref, acc_ref, o_ref):
    o_ref[...] = w_ref[...]


def kernel(weight):
    wt = weight.T
    out_sc = _sc_copy(wt)
    out_t = pl.pallas_call(
        _tc_body,
        out_shape=jax.ShapeDtypeStruct((_EMBED_DIM, _NUM_NODES), jnp.float32),
        grid=(_TC_GRID,),
        in_specs=[
            pl.BlockSpec((_EMBED_DIM, _BC),
                         lambda i: (0, i + _SC_COLS // _BC)),
            pl.BlockSpec(memory_space=pl.ANY),
        ],
        out_specs=pl.BlockSpec((_EMBED_DIM, _BC),
                               lambda i: (0, i + _SC_COLS // _BC)),
        input_output_aliases={1: 0},
    )(wt, out_sc)
    return out_t.T


# speedup vs baseline: 1.3636x; 1.0172x over previous
import functools

import jax
import jax.numpy as jnp
from jax import lax
from jax.experimental import pallas as pl
from jax.experimental.pallas import tpu as pltpu
from jax.experimental.pallas import tpu_sc as plsc

_NUM_NODES = 1000000
_EMBED_DIM = 64
_NUM_CORES = 2
_NUM_SUBCORES = 16
_NUM_WORKERS = _NUM_CORES * _NUM_SUBCORES

_BC = 32768
_SC_COLS = 393216
_NQ = 4
_COLS_PER_W = _SC_COLS // _NQ
_CHUNK = 3072
_NCHUNKS = _COLS_PER_W // _CHUNK
_NBUF = 4
_NBANDS = 8

_TC_GRID = (_NUM_NODES - _SC_COLS + _BC - 1) // _BC

_MESH = plsc.VectorSubcoreMesh(core_axis_name="c", subcore_axis_name="s")


@functools.partial(
    pl.kernel,
    out_type=jax.ShapeDtypeStruct((_EMBED_DIM, _NUM_NODES), jnp.float32),
    mesh=_MESH,
    scratch_types=[
        [pltpu.VMEM((8, _CHUNK), jnp.float32) for _ in range(_NBUF)],
        [pltpu.SemaphoreType.DMA for _ in range(_NBUF)],
        [pltpu.SemaphoreType.DMA for _ in range(_NBUF)],
    ],
)
def _sc_copy(w_hbm, o_hbm, bufs, in_sems, out_sems):
    wid = lax.axis_index("s") * _NUM_CORES + lax.axis_index("c")
    band = wid // _NQ
    row = pl.multiple_of(band * 8, 8)
    cbase = pl.multiple_of((wid % _NQ) * _COLS_PER_W, 128)

    def _in_copy(k, b):
        off = pl.multiple_of(cbase + k * _CHUNK, 128)
        return pltpu.make_async_copy(
            w_hbm.at[pl.ds(row, 8), pl.ds(off, _CHUNK)], bufs[b], in_sems[b])

    def _out_copy(k, b):
        off = pl.multiple_of(cbase + k * _CHUNK, 128)
        return pltpu.make_async_copy(
            bufs[b], o_hbm.at[pl.ds(row, 8), pl.ds(off, _CHUNK)], out_sems[b])

    for j in range(_NBUF - 1):
        _in_copy(j, j).start()
    for k in range(_NCHUNKS):
        b = k % _NBUF
        _in_copy(k, b).wait()
        _out_copy(k, b).start()
        if k + _NBUF - 1 < _NCHUNKS:
            if k >= 1:
                _out_copy(k - 1, (k + _NBUF - 1) % _NBUF).wait()
            _in_copy(k + _NBUF - 1, (k + _NBUF - 1) % _NBUF).start()
    for k in range(_NCHUNKS - _NBUF, _NCHUNKS):
        _out_copy(k, k % _NBUF).wait()


def _tc_body(w_ref, acc_ref, o_ref):
    o_ref[...] = w_ref[...]


def kernel(weight):
    wt = weight.T
    out_sc = _sc_copy(wt)
    out_t = pl.pallas_call(
        _tc_body,
        out_shape=jax.ShapeDtypeStruct((_EMBED_DIM, _NUM_NODES), jnp.float32),
        grid=(_TC_GRID,),
        in_specs=[
            pl.BlockSpec((_EMBED_DIM, _BC),
                         lambda i: (0, i + _SC_COLS // _BC)),
            pl.BlockSpec(memory_space=pl.ANY),
        ],
        out_specs=pl.BlockSpec((_EMBED_DIM, _BC),
                               lambda i: (0, i + _SC_COLS // _BC)),
        input_output_aliases={1: 0},
    )(wt, out_sc)
    return out_t.T


# speedup vs baseline: 1.3815x; 1.0131x over previous
import functools

import jax
import jax.numpy as jnp
from jax import lax
from jax.experimental import pallas as pl
from jax.experimental.pallas import tpu as pltpu
from jax.experimental.pallas import tpu_sc as plsc

_NUM_NODES = 1000000
_EMBED_DIM = 64
_NUM_CORES = 2
_NUM_SUBCORES = 16
_NUM_WORKERS = _NUM_CORES * _NUM_SUBCORES

_BC = 49152
_SC_COLS = 294912
_NQ = 4
_COLS_PER_W = _SC_COLS // _NQ
_CHUNK = 3072
_NCHUNKS = _COLS_PER_W // _CHUNK
_NBUF = 4
_NBANDS = 8

_TC_GRID = (_NUM_NODES - _SC_COLS + _BC - 1) // _BC

_MESH = plsc.VectorSubcoreMesh(core_axis_name="c", subcore_axis_name="s")


@functools.partial(
    pl.kernel,
    out_type=jax.ShapeDtypeStruct((_EMBED_DIM, _NUM_NODES), jnp.float32),
    mesh=_MESH,
    scratch_types=[
        [pltpu.VMEM((8, _CHUNK), jnp.float32) for _ in range(_NBUF)],
        [pltpu.SemaphoreType.DMA for _ in range(_NBUF)],
        [pltpu.SemaphoreType.DMA for _ in range(_NBUF)],
    ],
)
def _sc_copy(w_hbm, o_hbm, bufs, in_sems, out_sems):
    wid = lax.axis_index("s") * _NUM_CORES + lax.axis_index("c")
    band = wid // _NQ
    row = pl.multiple_of(band * 8, 8)
    cbase = pl.multiple_of((wid % _NQ) * _COLS_PER_W, 128)

    def _in_copy(k, b):
        off = pl.multiple_of(cbase + k * _CHUNK, 128)
        return pltpu.make_async_copy(
            w_hbm.at[pl.ds(row, 8), pl.ds(off, _CHUNK)], bufs[b], in_sems[b])

    def _out_copy(k, b):
        off = pl.multiple_of(cbase + k * _CHUNK, 128)
        return pltpu.make_async_copy(
            bufs[b], o_hbm.at[pl.ds(row, 8), pl.ds(off, _CHUNK)], out_sems[b])

    for j in range(_NBUF - 1):
        _in_copy(j, j).start()
    for k in range(_NCHUNKS):
        b = k % _NBUF
        _in_copy(k, b).wait()
        _out_copy(k, b).start()
        if k + _NBUF - 1 < _NCHUNKS:
            if k >= 1:
                _out_copy(k - 1, (k + _NBUF - 1) % _NBUF).wait()
            _in_copy(k + _NBUF - 1, (k + _NBUF - 1) % _NBUF).start()
    for k in range(_NCHUNKS - _NBUF, _NCHUNKS):
        _out_copy(k, k % _NBUF).wait()


def _tc_body(w_ref, acc_ref, o_ref):
    o_ref[...] = w_ref[...]


def kernel(weight):
    wt = weight.T
    out_sc = _sc_copy(wt)
    out_t = pl.pallas_call(
        _tc_body,
        out_shape=jax.ShapeDtypeStruct((_EMBED_DIM, _NUM_NODES), jnp.float32),
        grid=(_TC_GRID,),
        in_specs=[
            pl.BlockSpec((_EMBED_DIM, _BC),
                         lambda i: (0, i + _SC_COLS // _BC)),
            pl.BlockSpec(memory_space=pl.ANY),
        ],
        out_specs=pl.BlockSpec((_EMBED_DIM, _BC),
                               lambda i: (0, i + _SC_COLS // _BC)),
        input_output_aliases={1: 0},
    )(wt, out_sc)
    return out_t.T
